# Initial kernel scaffold; baseline (speedup 1.0000x reference)
import jax
import jax.numpy as jnp
from jax.experimental import pallas as pl


def kernel(memory_value):
    raise NotImplementedError("write your pallas kernel here")


# speedup vs baseline: 12.5895x; 12.5895x over previous
import jax
import jax.numpy as jnp
from jax.experimental import pallas as pl

_N = 4096
_D = 256
_K = 33
_R = 256


def _topk_body(xrow_ref, xall_ref, rows_ref, cols_ref, vals_ref):
    xr = xrow_ref[...]
    xa = xall_ref[...]

    na = jnp.sqrt(jnp.sum(xa * xa, axis=1, keepdims=True))
    ba = xa / jnp.maximum(na, 1e-6)
    nr = jnp.sqrt(jnp.sum(xr * xr, axis=1, keepdims=True))
    br = xr / jnp.maximum(nr, 1e-6)

    sim = jax.lax.dot_general(
        br, ba, (((1,), (1,)), ((), ())), preferred_element_type=jnp.float32
    )

    col_iota = jax.lax.broadcasted_iota(jnp.int32, (_R, _N), 1)
    s = sim
    idxs = []
    vals = []
    for _ in range(_K):
        m = jnp.max(s, axis=1)
        hit = s == m[:, None]
        idx = jnp.min(jnp.where(hit, col_iota, _N), axis=1)
        idxs.append(idx)
        vals.append(m)
        s = jnp.where(col_iota == idx[:, None], -jnp.inf, s)

    idx = jnp.stack(idxs, axis=1)
    val = jnp.stack(vals, axis=1)

    rank = jnp.zeros((_R, _K), jnp.int32)
    for t in range(_K):
        rank = rank + (idx[:, t : t + 1] < idx).astype(jnp.int32)
    lane = jax.lax.broadcasted_iota(jnp.int32, (_R, _K), 1)
    scol = jnp.zeros((_R, _K), jnp.int32)
    sval = jnp.zeros((_R, _K), jnp.float32)
    for t in range(_K):
        onehot = rank[:, t : t + 1] == lane
        scol = jnp.where(onehot, idx[:, t : t + 1], scol)
        sval = jnp.where(onehot, val[:, t : t + 1], sval)

    i = pl.program_id(0)
    rows_ref[...] = jax.lax.broadcasted_iota(jnp.int32, (_R, _K), 0) + i * _R
    cols_ref[...] = scol
    vals_ref[...] = sval


def kernel(memory_value):
    grid = _N // _R
    rows, cols, vals = pl.pallas_call(
        _topk_body,
        grid=(grid,),
        in_specs=[
            pl.BlockSpec((_R, _D), lambda i: (i, 0)),
            pl.BlockSpec((_N, _D), lambda i: (0, 0)),
        ],
        out_specs=[
            pl.BlockSpec((_R, _K), lambda i: (i, 0)),
            pl.BlockSpec((_R, _K), lambda i: (i, 0)),
            pl.BlockSpec((_R, _K), lambda i: (i, 0)),
        ],
        out_shape=[
            jax.ShapeDtypeStruct((_N, _K), jnp.int32),
            jax.ShapeDtypeStruct((_N, _K), jnp.int32),
            jax.ShapeDtypeStruct((_N, _K), jnp.float32),
        ],
    )(memory_value, memory_value)

    edge_index = jnp.stack([rows.reshape(-1), cols.reshape(-1)]).astype(jnp.int64)
    edge_weight = vals.reshape(-1)
    return (edge_index, edge_weight)
